# baseline (speedup 1.0000x reference)
import jax
import jax.numpy as jnp
from jax.experimental import pallas as pl
from jax.experimental.pallas import tpu as pltpu

_HW = 5184
_BD = 8


def _roll_off(o):
    dz, rem = o // 9, o % 9
    dy, dx = rem // 3, rem % 3
    return dz, (dy - 1) * 72 + (dx - 1)


def _conv_kernel(in_ref, bt1, bb1, bt2, bb2, bt3, bb3, bt4, bb4,
                 out_ref, x0_s, m_s, x1_s, x2_s, x3_s, a_s):
    d = pl.program_id(1)

    slab = in_ref[0, pl.ds(d * _BD, 16), :]
    m = (slab != 0.0).astype(jnp.float32)
    m_s[...] = m
    x0_s[...] = (slab * 0.5 + 0.5) * m

    def layer(src_ref, cin, bt_ref, bias_ref, cout, n_out, layer_idx, store):
        K = 27 * cin

        def body(j, carry):
            for o in range(27):
                dz, off = _roll_off(o)
                if cin == 1:
                    plane = src_ref[j + dz][None, :]
                else:
                    plane = src_ref[j + dz]
                if off:
                    plane = jnp.roll(plane, -off, axis=-1)
                a_s[pl.ds(o * cin, cin), :] = plane
            y = jax.lax.dot_general(
                bt_ref[...], a_s[pl.ds(0, K), :],
                dimension_numbers=(((1,), (0,)), ((), ())),
                preferred_element_type=jnp.float32,
                precision=jax.lax.Precision.HIGHEST)
            y = y + bias_ref[...]
            y = jnp.maximum(y, 0.0)
            y = y * m_s[layer_idx + j][None, :]
            store(j, y)
            return carry

        jax.lax.fori_loop(0, n_out, body, 0)

    def store1(j, y):
        x1_s[j, :, :] = y

    def store2(j, y):
        x2_s[j, :, :] = y

    def store3(j, y):
        x3_s[j, :, :] = y

    def store4(j, y):
        out_ref[0, pl.ds(j, 1), :] = y * 2.0 - 1.0

    layer(x0_s, 1, bt1, bb1, 16, 14, 1, store1)
    layer(x1_s, 16, bt2, bb2, 32, 12, 2, store2)
    layer(x2_s, 32, bt3, bb3, 16, 10, 3, store3)
    layer(x3_s, 16, bt4, bb4, 1, _BD, 4, store4)


def _prep_w(W):
    O, I = W.shape[0], W.shape[1]
    return W.reshape(O, I, 27).transpose(0, 2, 1).reshape(O, 27 * I)


@jax.jit
def kernel(inputTSDF, W1, b1, W2, b2, W3, b3, W4, b4):
    x = inputTSDF[:, 0]
    xp = jnp.pad(x, ((0, 0), (4, 4), (4, 4), (4, 4)))
    xp = xp.reshape(2, 72, _HW)

    args = (xp,
            _prep_w(W1), b1.reshape(-1, 1),
            _prep_w(W2), b2.reshape(-1, 1),
            _prep_w(W3), b3.reshape(-1, 1),
            _prep_w(W4), b4.reshape(-1, 1))

    small = lambda shp: pl.BlockSpec(shp, lambda b, d: (0, 0))
    out = pl.pallas_call(
        _conv_kernel,
        grid=(2, 64 // _BD),
        in_specs=[
            pl.BlockSpec((1, 72, _HW), lambda b, d: (b, 0, 0)),
            small((16, 27)), small((16, 1)),
            small((32, 432)), small((32, 1)),
            small((16, 864)), small((16, 1)),
            small((1, 432)), small((1, 1)),
        ],
        out_specs=pl.BlockSpec((1, _BD, _HW), lambda b, d: (b, d, 0)),
        out_shape=jax.ShapeDtypeStruct((2, 64, _HW), jnp.float32),
        scratch_shapes=[
            pltpu.VMEM((16, _HW), jnp.float32),
            pltpu.VMEM((16, _HW), jnp.float32),
            pltpu.VMEM((14, 16, _HW), jnp.float32),
            pltpu.VMEM((12, 32, _HW), jnp.float32),
            pltpu.VMEM((10, 16, _HW), jnp.float32),
            pltpu.VMEM((864, _HW), jnp.float32),
        ],
        compiler_params=pltpu.CompilerParams(
            dimension_semantics=("arbitrary", "arbitrary")),
    )(*args)

    res = out.reshape(2, 64, 72, 72)[:, :, 4:68, 4:68]
    return res.reshape(2, 1, 64, 64, 64)


# speedup vs baseline: 1.5057x; 1.5057x over previous
import jax
import jax.numpy as jnp
from jax.experimental import pallas as pl
from jax.experimental.pallas import tpu as pltpu

_HW = 5184
_BD = 8


def _offs():
    out = []
    for q in range(9):
        dy, dx = q // 3, q % 3
        out.append((dy - 1) * 72 + (dx - 1))
    return out


_OFFS = _offs()


def _conv_kernel(in_ref, bs1, bb1, bs2, bb2, bs3, bb3, bs4, bb4,
                 out_ref, x0_s, m_s, x1_s, x2_s, x3_s, ah_s, al_s):
    d = pl.program_id(1)

    slab = in_ref[0, pl.ds(d * _BD, 16), :]
    m = (slab != 0.0).astype(jnp.float32)
    m_s[...] = m
    x0_s[...] = (slab * 0.5 + 0.5) * m

    def layer(src, cin, stride, bs_ref, bias_ref, n_out, layer_idx, store):
        def fill(p, slot):
            if cin == 1:
                plane = src[p, :][None, :]
            else:
                plane = src[p]
            for q, off in enumerate(_OFFS):
                v = jnp.roll(plane, -off, axis=-1) if off else plane
                vh = v.astype(jnp.bfloat16)
                ah_s[pl.ds(slot * stride + q * cin, cin), :] = vh
                al_s[pl.ds(slot * stride + q * cin, cin), :] = (
                    v - vh.astype(jnp.float32)).astype(jnp.bfloat16)

        fill(0, 0)
        fill(1, 1)

        def dot(lhs, rhs):
            return jax.lax.dot_general(
                lhs, rhs, dimension_numbers=(((1,), (0,)), ((), ())),
                preferred_element_type=jnp.float32)

        def body(j, carry):
            fill(j + 2, (j + 2) % 3)
            B = bs_ref[j % 3]
            Bh = B.astype(jnp.bfloat16)
            Bl = (B - Bh.astype(jnp.float32)).astype(jnp.bfloat16)
            Ah = ah_s[pl.ds(0, 3 * stride), :]
            Al = al_s[pl.ds(0, 3 * stride), :]
            y = dot(Bh, Ah) + dot(Bh, Al) + dot(Bl, Ah)
            y = y + bias_ref[...]
            y = jnp.maximum(y, 0.0)
            y = y * m_s[layer_idx + j][None, :]
            store(j, y)
            return carry

        jax.lax.fori_loop(0, n_out, body, 0)

    x0v = x0_s[...]
    for q, off in enumerate(_OFFS):
        v = jnp.roll(x0v, -off, axis=-1) if off else x0v
        vh = v.astype(jnp.bfloat16)
        ah_s[pl.ds(q * 16, 16), :] = vh
        al_s[pl.ds(q * 16, 16), :] = (v - vh.astype(jnp.float32)).astype(
            jnp.bfloat16)

    def dot0(lhs, rhs):
        return jax.lax.dot_general(
            lhs, rhs, dimension_numbers=(((1,), (0,)), ((), ())),
            preferred_element_type=jnp.float32)

    def body1(j, carry):
        B = bs1[j]
        Bh = B.astype(jnp.bfloat16)
        Bl = (B - Bh.astype(jnp.float32)).astype(jnp.bfloat16)
        Ah = ah_s[pl.ds(0, 144), :]
        Al = al_s[pl.ds(0, 144), :]
        y = dot0(Bh, Ah) + dot0(Bh, Al) + dot0(Bl, Ah)
        y = y + bb1[...]
        y = jnp.maximum(y, 0.0)
        y = y * m_s[1 + j][None, :]
        x1_s[j, :, :] = y
        return carry

    jax.lax.fori_loop(0, 14, body1, 0)

    layer(x1_s, 16, 144, bs2, bb2, 12, 2,
          lambda j, y: x2_s.__setitem__((j,), y))
    layer(x2_s, 32, 288, bs3, bb3, 10, 3,
          lambda j, y: x3_s.__setitem__((j,), y))

    def store4(j, y):
        out_ref[0, pl.ds(j, 1), :] = y * 2.0 - 1.0

    layer(x3_s, 16, 144, bs4, bb4, _BD, 4, store4)


import numpy as _np

_OH1 = _np.zeros((14, 3, 16), _np.float32)
for _j in range(14):
    for _dz in range(3):
        _OH1[_j, _dz, _j + _dz] = 1.0


def _prep_b1(W1):
    Wf = W1.reshape(16, 3, 9)
    return jnp.einsum('cdq,jdp->jcqp', Wf, _OH1).reshape(14, 16, 144)


def _prep_bstack(W, stride):
    O, I = W.shape[0], W.shape[1]
    Wf = W.reshape(O, I, 3, 9)
    rots = []
    for r in range(3):
        blocks = []
        for s in range(3):
            dz = (s - r) % 3
            blk = Wf[:, :, dz, :].transpose(0, 2, 1).reshape(O, 9 * I)
            if 9 * I < stride:
                blk = jnp.pad(blk, ((0, 0), (0, stride - 9 * I)))
            blocks.append(blk)
        rots.append(jnp.concatenate(blocks, axis=1))
    return jnp.stack(rots)


@jax.jit
def kernel(inputTSDF, W1, b1, W2, b2, W3, b3, W4, b4):
    x = inputTSDF[:, 0]
    xp = jnp.pad(x, ((0, 0), (4, 4), (4, 4), (4, 4)))
    xp = xp.reshape(2, 72, _HW)

    args = (xp,
            _prep_b1(W1), b1.reshape(-1, 1),
            _prep_bstack(W2, 144), b2.reshape(-1, 1),
            _prep_bstack(W3, 288), b3.reshape(-1, 1),
            _prep_bstack(W4, 144), b4.reshape(-1, 1))

    small = lambda shp: pl.BlockSpec(shp, lambda b, d: tuple(0 for _ in shp))
    out = pl.pallas_call(
        _conv_kernel,
        grid=(2, 64 // _BD),
        in_specs=[
            pl.BlockSpec((1, 72, _HW), lambda b, d: (b, 0, 0)),
            small((14, 16, 144)), small((16, 1)),
            small((3, 32, 432)), small((32, 1)),
            small((3, 16, 864)), small((16, 1)),
            small((3, 1, 432)), small((1, 1)),
        ],
        out_specs=pl.BlockSpec((1, _BD, _HW), lambda b, d: (b, d, 0)),
        out_shape=jax.ShapeDtypeStruct((2, 64, _HW), jnp.float32),
        scratch_shapes=[
            pltpu.VMEM((16, _HW), jnp.float32),
            pltpu.VMEM((16, _HW), jnp.float32),
            pltpu.VMEM((14, 16, _HW), jnp.float32),
            pltpu.VMEM((12, 32, _HW), jnp.float32),
            pltpu.VMEM((10, 16, _HW), jnp.float32),
            pltpu.VMEM((864, _HW), jnp.bfloat16),
            pltpu.VMEM((864, _HW), jnp.bfloat16),
        ],
        compiler_params=pltpu.CompilerParams(
            dimension_semantics=("arbitrary", "arbitrary")),
    )(*args)

    res = out.reshape(2, 64, 72, 72)[:, :, 4:68, 4:68]
    return res.reshape(2, 1, 64, 64, 64)


# speedup vs baseline: 1.6945x; 1.1254x over previous
import jax
import jax.numpy as jnp
from jax.experimental import pallas as pl
from jax.experimental.pallas import tpu as pltpu

_HW = 5184
_BD = 8


def _offs():
    out = []
    for q in range(9):
        dy, dx = q // 3, q % 3
        out.append((dy - 1) * 72 + (dx - 1))
    return out


_OFFS = _offs()


def _conv_kernel(in_ref, bs1, bb1, bs2, bb2, bs3, bb3, bs4, bb4,
                 out_ref, x0_s, m_s, x1h_s, x1l_s, x2h_s, x2l_s,
                 x3h_s, x3l_s, ah_s, al_s):
    d = pl.program_id(1)

    slab = in_ref[0, pl.ds(d * _BD, 16), :]
    m = (slab != 0.0).astype(jnp.float32)
    m_s[...] = m
    x0_s[...] = (slab * 0.5 + 0.5) * m

    def layer(srch, srcl, cin, stride, bs_ref, bias_ref, n_out, layer_idx,
              store):
        def fill(p, slot):
            ph = srch[p]
            plo = srcl[p]
            for q, off in enumerate(_OFFS):
                ah_s[pl.ds(slot * stride + q * cin, cin), :] = (
                    jnp.roll(ph, -off, axis=-1) if off else ph)
                al_s[pl.ds(slot * stride + q * cin, cin), :] = (
                    jnp.roll(plo, -off, axis=-1) if off else plo)

        fill(0, 0)
        fill(1, 1)

        def dot(lhs, rhs):
            return jax.lax.dot_general(
                lhs, rhs, dimension_numbers=(((1,), (0,)), ((), ())),
                preferred_element_type=jnp.float32)

        def body(j, carry):
            fill(j + 2, (j + 2) % 3)
            B = bs_ref[j % 3]
            Bh = B.astype(jnp.bfloat16)
            Bl = (B - Bh.astype(jnp.float32)).astype(jnp.bfloat16)
            Ah = ah_s[pl.ds(0, 3 * stride), :]
            Al = al_s[pl.ds(0, 3 * stride), :]
            y = dot(Bh, Ah) + dot(Bh, Al) + dot(Bl, Ah)
            y = y + bias_ref[...]
            y = jnp.maximum(y, 0.0)
            y = y * m_s[layer_idx + j][None, :]
            store(j, y)
            return carry

        jax.lax.fori_loop(0, n_out, body, 0)

    x0v = x0_s[...]
    x0h = x0v.astype(jnp.bfloat16)
    x0l = (x0v - x0h.astype(jnp.float32)).astype(jnp.bfloat16)
    for q, off in enumerate(_OFFS):
        ah_s[pl.ds(q * 16, 16), :] = (
            jnp.roll(x0h, -off, axis=-1) if off else x0h)
        al_s[pl.ds(q * 16, 16), :] = (
            jnp.roll(x0l, -off, axis=-1) if off else x0l)

    def dot0(lhs, rhs):
        return jax.lax.dot_general(
            lhs, rhs, dimension_numbers=(((1,), (0,)), ((), ())),
            preferred_element_type=jnp.float32)

    def body1(j, carry):
        B = bs1[j]
        Bh = B.astype(jnp.bfloat16)
        Bl = (B - Bh.astype(jnp.float32)).astype(jnp.bfloat16)
        Ah = ah_s[pl.ds(0, 144), :]
        Al = al_s[pl.ds(0, 144), :]
        y = dot0(Bh, Ah) + dot0(Bh, Al) + dot0(Bl, Ah)
        y = y + bb1[...]
        y = jnp.maximum(y, 0.0)
        y = y * m_s[1 + j][None, :]
        yh = y.astype(jnp.bfloat16)
        x1h_s[j, :, :] = yh
        x1l_s[j, :, :] = (y - yh.astype(jnp.float32)).astype(jnp.bfloat16)
        return carry

    jax.lax.fori_loop(0, 14, body1, 0)

    def split_store(hs, ls):
        def store(j, y):
            yh = y.astype(jnp.bfloat16)
            hs[j, :, :] = yh
            ls[j, :, :] = (y - yh.astype(jnp.float32)).astype(jnp.bfloat16)
        return store

    layer(x1h_s, x1l_s, 16, 144, bs2, bb2, 12, 2, split_store(x2h_s, x2l_s))
    layer(x2h_s, x2l_s, 32, 288, bs3, bb3, 10, 3, split_store(x3h_s, x3l_s))

    def store4(j, y):
        out_ref[0, pl.ds(j, 1), :] = y * 2.0 - 1.0

    layer(x3h_s, x3l_s, 16, 144, bs4, bb4, _BD, 4, store4)


import numpy as _np

_OH1 = _np.zeros((14, 3, 16), _np.float32)
for _j in range(14):
    for _dz in range(3):
        _OH1[_j, _dz, _j + _dz] = 1.0


def _prep_b1(W1):
    Wf = W1.reshape(16, 3, 9)
    return jnp.einsum('cdq,jdp->jcqp', Wf, _OH1).reshape(14, 16, 144)


def _prep_bstack(W, stride):
    O, I = W.shape[0], W.shape[1]
    Wf = W.reshape(O, I, 3, 9)
    rots = []
    for r in range(3):
        blocks = []
        for s in range(3):
            dz = (s - r) % 3
            blk = Wf[:, :, dz, :].transpose(0, 2, 1).reshape(O, 9 * I)
            if 9 * I < stride:
                blk = jnp.pad(blk, ((0, 0), (0, stride - 9 * I)))
            blocks.append(blk)
        rots.append(jnp.concatenate(blocks, axis=1))
    return jnp.stack(rots)


@jax.jit
def kernel(inputTSDF, W1, b1, W2, b2, W3, b3, W4, b4):
    x = inputTSDF[:, 0]
    xp = jnp.pad(x, ((0, 0), (4, 4), (4, 4), (4, 4)))
    xp = xp.reshape(2, 72, _HW)

    args = (xp,
            _prep_b1(W1), b1.reshape(-1, 1),
            _prep_bstack(W2, 144), b2.reshape(-1, 1),
            _prep_bstack(W3, 288), b3.reshape(-1, 1),
            _prep_bstack(W4, 144), b4.reshape(-1, 1))

    small = lambda shp: pl.BlockSpec(shp, lambda b, d: tuple(0 for _ in shp))
    out = pl.pallas_call(
        _conv_kernel,
        grid=(2, 64 // _BD),
        in_specs=[
            pl.BlockSpec((1, 72, _HW), lambda b, d: (b, 0, 0)),
            small((14, 16, 144)), small((16, 1)),
            small((3, 32, 432)), small((32, 1)),
            small((3, 16, 864)), small((16, 1)),
            small((3, 1, 432)), small((1, 1)),
        ],
        out_specs=pl.BlockSpec((1, _BD, _HW), lambda b, d: (b, d, 0)),
        out_shape=jax.ShapeDtypeStruct((2, 64, _HW), jnp.float32),
        scratch_shapes=[
            pltpu.VMEM((16, _HW), jnp.float32),
            pltpu.VMEM((16, _HW), jnp.float32),
            pltpu.VMEM((14, 16, _HW), jnp.bfloat16),
            pltpu.VMEM((14, 16, _HW), jnp.bfloat16),
            pltpu.VMEM((12, 32, _HW), jnp.bfloat16),
            pltpu.VMEM((12, 32, _HW), jnp.bfloat16),
            pltpu.VMEM((10, 16, _HW), jnp.bfloat16),
            pltpu.VMEM((10, 16, _HW), jnp.bfloat16),
            pltpu.VMEM((864, _HW), jnp.bfloat16),
            pltpu.VMEM((864, _HW), jnp.bfloat16),
        ],
        compiler_params=pltpu.CompilerParams(
            dimension_semantics=("arbitrary", "arbitrary")),
    )(*args)

    res = out.reshape(2, 64, 72, 72)[:, :, 4:68, 4:68]
    return res.reshape(2, 1, 64, 64, 64)


# speedup vs baseline: 2.1103x; 1.2454x over previous
import numpy as _np

import jax
import jax.numpy as jnp
from jax.experimental import pallas as pl
from jax.experimental.pallas import tpu as pltpu

_HW = 5184
_BD = 8

_OFFS = [(dy - 1) * 72 + (dx - 1) for dy in range(3) for dx in range(3)]
_CHUNKS = [(0, 2560), (2560, 2624)]


def _conv_kernel(in_ref, b1h, b1l, bb1, bs2h, bs2l, bb2,
                 bs3h, bs3l, bb3, bs4h, bs4l, bb4,
                 out_ref, x0_s, m_s, x1h_s, x1l_s, x2h_s, x2l_s,
                 x3h_s, x3l_s, ah_s, al_s):
    d = pl.program_id(1)

    slab = in_ref[0, pl.ds(d * _BD, 16), :]
    m = (slab != 0.0).astype(jnp.float32)
    m_s[...] = m
    x0_s[...] = (slab * 0.5 + 0.5) * m

    def dot(lhs, rhs):
        return jax.lax.dot_general(
            lhs, rhs, dimension_numbers=(((1,), (0,)), ((), ())),
            preferred_element_type=jnp.float32)

    def dot3(Bh, Bl, K, c0, csz):
        Ah = ah_s[pl.ds(0, K), pl.ds(c0, csz)]
        Al = al_s[pl.ds(0, K), pl.ds(c0, csz)]
        return dot(Bh, Ah) + dot(Bh, Al) + dot(Bl, Ah)

    x0v = x0_s[...]
    x0h = x0v.astype(jnp.bfloat16)
    x0l = (x0v - x0h.astype(jnp.float32)).astype(jnp.bfloat16)
    for q, off in enumerate(_OFFS):
        ah_s[pl.ds(q * 16, 16), :] = (
            jnp.roll(x0h, -off, axis=-1) if off else x0h)
        al_s[pl.ds(q * 16, 16), :] = (
            jnp.roll(x0l, -off, axis=-1) if off else x0l)

    for c0, csz in _CHUNKS:
        Y = dot3(b1h[...], b1l[...], 144, c0, csz)
        Y = jnp.maximum(Y + jnp.tile(bb1[...], (14, 1)), 0.0)
        for j in range(14):
            y = Y[j * 16:(j + 1) * 16] * m_s[1 + j, pl.ds(c0, csz)][None, :]
            yh = y.astype(jnp.bfloat16)
            x1h_s[j, :, pl.ds(c0, csz)] = yh
            x1l_s[j, :, pl.ds(c0, csz)] = (
                y - yh.astype(jnp.float32)).astype(jnp.bfloat16)

    def layer(srch, srcl, cin, bs_h, bs_l, bias_ref, cout, n_out,
              layer_idx, store):
        stride = 9 * cin

        def fill(p, slot):
            ph = srch[p]
            plo = srcl[p]
            for q, off in enumerate(_OFFS):
                r = slot * stride + q * cin
                ah_s[pl.ds(r, cin), :] = (
                    jnp.roll(ph, -off, axis=-1) if off else ph)
                al_s[pl.ds(r, cin), :] = (
                    jnp.roll(plo, -off, axis=-1) if off else plo)

        fill(0, 0)
        fill(1, 1)
        bias2 = jnp.tile(bias_ref[...], (2, 1))

        def body(j2, carry):
            fill(2 * j2 + 2, (2 * j2 + 2) % 4)
            fill(2 * j2 + 3, (2 * j2 + 3) % 4)
            Bh = bs_h[j2 % 2]
            Bl = bs_l[j2 % 2]
            for c0, csz in _CHUNKS:
                Y = dot3(Bh, Bl, 4 * stride, c0, csz)
                Y = jnp.maximum(Y + bias2, 0.0)
                for j_rel in range(2):
                    jj = 2 * j2 + j_rel
                    y = (Y[j_rel * cout:(j_rel + 1) * cout]
                         * m_s[layer_idx + jj, pl.ds(c0, csz)][None, :])
                    store(jj, y, c0, csz)
            return carry

        jax.lax.fori_loop(0, n_out // 2, body, 0)

    def split_store(hs, ls):
        def store(j, y, c0, csz):
            yh = y.astype(jnp.bfloat16)
            hs[j, :, pl.ds(c0, csz)] = yh
            ls[j, :, pl.ds(c0, csz)] = (
                y - yh.astype(jnp.float32)).astype(jnp.bfloat16)
        return store

    layer(x1h_s, x1l_s, 16, bs2h, bs2l, bb2, 32, 12, 2,
          split_store(x2h_s, x2l_s))
    layer(x2h_s, x2l_s, 32, bs3h, bs3l, bb3, 16, 10, 3,
          split_store(x3h_s, x3l_s))

    def store4(j, y, c0, csz):
        out_ref[0, pl.ds(j, 1), pl.ds(c0, csz)] = y * 2.0 - 1.0

    layer(x3h_s, x3l_s, 16, bs4h, bs4l, bb4, 1, 8, 4, store4)


def _split(B):
    Bh = B.astype(jnp.bfloat16)
    Bl = (B - Bh.astype(jnp.float32)).astype(jnp.bfloat16)
    return Bh, Bl


_OH1 = _np.zeros((14, 3, 16), _np.float32)
for _j in range(14):
    for _dz in range(3):
        _OH1[_j, _dz, _j + _dz] = 1.0

_OH2 = _np.zeros((2, 2, 3, 4), _np.float32)
for _v in range(2):
    for _jr in range(2):
        for _dz in range(3):
            _OH2[_v, _jr, _dz, (2 * _v + _jr + _dz) % 4] = 1.0


def _prep_b1(W1):
    Wf = W1.reshape(16, 3, 9)
    B = jnp.einsum('cdq,jdp->jcqp', Wf, _OH1).reshape(224, 144)
    return _split(B)


def _prep_ring(W):
    O, I = W.shape[0], W.shape[1]
    Wf = W.reshape(O, I, 3, 9)
    B = jnp.einsum('cidq,vjds->vjcsqi', Wf, _OH2).reshape(2, 2 * O, 36 * I)
    return _split(B)


@jax.jit
def kernel(inputTSDF, W1, b1, W2, b2, W3, b3, W4, b4):
    x = inputTSDF[:, 0]
    xp = jnp.pad(x, ((0, 0), (4, 4), (4, 4), (4, 4)))
    xp = xp.reshape(2, 72, _HW)

    b1h, b1l = _prep_b1(W1)
    bs2h, bs2l = _prep_ring(W2)
    bs3h, bs3l = _prep_ring(W3)
    bs4h, bs4l = _prep_ring(W4)

    args = (xp, b1h, b1l, b1.reshape(-1, 1),
            bs2h, bs2l, b2.reshape(-1, 1),
            bs3h, bs3l, b3.reshape(-1, 1),
            bs4h, bs4l, b4.reshape(-1, 1))

    small = lambda a: pl.BlockSpec(a.shape,
                                   lambda b, d: tuple(0 for _ in a.shape))
    in_specs = [pl.BlockSpec((1, 72, _HW), lambda b, d: (b, 0, 0))]
    in_specs += [small(a) for a in args[1:]]

    out = pl.pallas_call(
        _conv_kernel,
        grid=(2, 64 // _BD),
        in_specs=in_specs,
        out_specs=pl.BlockSpec((1, _BD, _HW), lambda b, d: (b, d, 0)),
        out_shape=jax.ShapeDtypeStruct((2, 64, _HW), jnp.float32),
        scratch_shapes=[
            pltpu.VMEM((16, _HW), jnp.float32),
            pltpu.VMEM((16, _HW), jnp.float32),
            pltpu.VMEM((14, 16, _HW), jnp.bfloat16),
            pltpu.VMEM((14, 16, _HW), jnp.bfloat16),
            pltpu.VMEM((12, 32, _HW), jnp.bfloat16),
            pltpu.VMEM((12, 32, _HW), jnp.bfloat16),
            pltpu.VMEM((10, 16, _HW), jnp.bfloat16),
            pltpu.VMEM((10, 16, _HW), jnp.bfloat16),
            pltpu.VMEM((1152, _HW), jnp.bfloat16),
            pltpu.VMEM((1152, _HW), jnp.bfloat16),
        ],
        compiler_params=pltpu.CompilerParams(
            dimension_semantics=("arbitrary", "arbitrary")),
    )(*args)

    res = out.reshape(2, 64, 72, 72)[:, :, 4:68, 4:68]
    return res.reshape(2, 1, 64, 64, 64)


# speedup vs baseline: 2.2500x; 1.0662x over previous
import numpy as _np

import jax
import jax.numpy as jnp
from jax.experimental import pallas as pl
from jax.experimental.pallas import tpu as pltpu

_HW = 5184
_BD = 8

_OFFS = [(dy - 1) * 72 + (dx - 1) for dy in range(3) for dx in range(3)]
_CHUNKS = [(0, 2560), (2560, 2624)]


def _conv_kernel(in_ref, b1h, b1l, bb1, bs2h, bs2l, bb2,
                 bs3h, bs3l, bb3, bs4h, bs4l, bb4,
                 out_ref, x0_s, m_s, x1h_s, x1l_s, x2h_s, x2l_s,
                 x3h_s, x3l_s, ah_s, al_s):
    d = pl.program_id(1)

    slab = in_ref[0, pl.ds(d * _BD, 16), :]
    m = (slab != 0.0).astype(jnp.float32)
    m_s[...] = m
    x0_s[...] = (slab * 0.5 + 0.5) * m

    def dot(lhs, rhs):
        return jax.lax.dot_general(
            lhs, rhs, dimension_numbers=(((1,), (0,)), ((), ())),
            preferred_element_type=jnp.float32)

    def dot3(Bh, Bl, K, c0, csz):
        Ah = ah_s[pl.ds(0, K), pl.ds(c0, csz)]
        Al = al_s[pl.ds(0, K), pl.ds(c0, csz)]
        return dot(Bh, Ah) + dot(Bh, Al) + dot(Bl, Ah)

    x0v = x0_s[...]
    x0h = x0v.astype(jnp.bfloat16)
    x0l = (x0v - x0h.astype(jnp.float32)).astype(jnp.bfloat16)
    for q, off in enumerate(_OFFS):
        ah_s[pl.ds(q * 16, 16), :] = (
            jnp.roll(x0h, -off, axis=-1) if off else x0h)
        al_s[pl.ds(q * 16, 16), :] = (
            jnp.roll(x0l, -off, axis=-1) if off else x0l)

    for c0, csz in _CHUNKS:
        Y = dot3(b1h[...], b1l[...], 144, c0, csz)
        Y = jnp.maximum(Y + jnp.tile(bb1[...], (14, 1)), 0.0)
        for j in range(14):
            y = Y[j * 16:(j + 1) * 16] * m_s[1 + j, pl.ds(c0, csz)][None, :]
            yh = y.astype(jnp.bfloat16)
            x1h_s[j, :, pl.ds(c0, csz)] = yh
            x1l_s[j, :, pl.ds(c0, csz)] = (
                y - yh.astype(jnp.float32)).astype(jnp.bfloat16)

    def layer(srch, srcl, cin, bs_h, bs_l, bias_ref, cout, n_out,
              layer_idx, store):
        stride = 9 * cin

        def fill(p, slot):
            ph = srch[p]
            plo = srcl[p]
            for q, off in enumerate(_OFFS):
                r = slot * stride + q * cin
                ah_s[pl.ds(r, cin), :] = (
                    jnp.roll(ph, -off, axis=-1) if off else ph)
                al_s[pl.ds(r, cin), :] = (
                    jnp.roll(plo, -off, axis=-1) if off else plo)

        fill(0, 0)
        fill(1, 1)
        bias2 = jnp.tile(bias_ref[...], (2, 1))

        def body(j2, carry):
            fill(2 * j2 + 2, (2 * j2 + 2) % 4)
            fill(2 * j2 + 3, (2 * j2 + 3) % 4)
            Bh = bs_h[j2 % 2]
            Bl = bs_l[j2 % 2]
            for c0, csz in _CHUNKS:
                Y = dot3(Bh, Bl, 4 * stride, c0, csz)
                Y = jnp.maximum(Y + bias2, 0.0)
                for j_rel in range(2):
                    jj = 2 * j2 + j_rel
                    y = (Y[j_rel * cout:(j_rel + 1) * cout]
                         * m_s[layer_idx + jj, pl.ds(c0, csz)][None, :])
                    store(jj, y, c0, csz)
            return carry

        jax.lax.fori_loop(0, n_out // 2, body, 0, unroll=2)

    def split_store(hs, ls):
        def store(j, y, c0, csz):
            yh = y.astype(jnp.bfloat16)
            hs[j, :, pl.ds(c0, csz)] = yh
            ls[j, :, pl.ds(c0, csz)] = (
                y - yh.astype(jnp.float32)).astype(jnp.bfloat16)
        return store

    layer(x1h_s, x1l_s, 16, bs2h, bs2l, bb2, 32, 12, 2,
          split_store(x2h_s, x2l_s))
    layer(x2h_s, x2l_s, 32, bs3h, bs3l, bb3, 16, 10, 3,
          split_store(x3h_s, x3l_s))

    def store4(j, y, c0, csz):
        out_ref[0, pl.ds(j, 1), pl.ds(c0, csz)] = y * 2.0 - 1.0

    layer(x3h_s, x3l_s, 16, bs4h, bs4l, bb4, 1, 8, 4, store4)


def _split(B):
    Bh = B.astype(jnp.bfloat16)
    Bl = (B - Bh.astype(jnp.float32)).astype(jnp.bfloat16)
    return Bh, Bl


_OH1 = _np.zeros((14, 3, 16), _np.float32)
for _j in range(14):
    for _dz in range(3):
        _OH1[_j, _dz, _j + _dz] = 1.0

_OH2 = _np.zeros((2, 2, 3, 4), _np.float32)
for _v in range(2):
    for _jr in range(2):
        for _dz in range(3):
            _OH2[_v, _jr, _dz, (2 * _v + _jr + _dz) % 4] = 1.0


def _prep_b1(W1):
    Wf = W1.reshape(16, 3, 9)
    B = jnp.einsum('cdq,jdp->jcqp', Wf, _OH1).reshape(224, 144)
    return _split(B)


def _prep_ring(W):
    O, I = W.shape[0], W.shape[1]
    Wf = W.reshape(O, I, 3, 9)
    B = jnp.einsum('cidq,vjds->vjcsqi', Wf, _OH2).reshape(2, 2 * O, 36 * I)
    return _split(B)


@jax.jit
def kernel(inputTSDF, W1, b1, W2, b2, W3, b3, W4, b4):
    x = inputTSDF[:, 0]
    xp = jnp.pad(x, ((0, 0), (4, 4), (4, 4), (4, 4)))
    xp = xp.reshape(2, 72, _HW)

    b1h, b1l = _prep_b1(W1)
    bs2h, bs2l = _prep_ring(W2)
    bs3h, bs3l = _prep_ring(W3)
    bs4h, bs4l = _prep_ring(W4)

    args = (xp, b1h, b1l, b1.reshape(-1, 1),
            bs2h, bs2l, b2.reshape(-1, 1),
            bs3h, bs3l, b3.reshape(-1, 1),
            bs4h, bs4l, b4.reshape(-1, 1))

    small = lambda a: pl.BlockSpec(a.shape,
                                   lambda b, d: tuple(0 for _ in a.shape))
    in_specs = [pl.BlockSpec((1, 72, _HW), lambda b, d: (b, 0, 0))]
    in_specs += [small(a) for a in args[1:]]

    out = pl.pallas_call(
        _conv_kernel,
        grid=(2, 64 // _BD),
        in_specs=in_specs,
        out_specs=pl.BlockSpec((1, _BD, _HW), lambda b, d: (b, d, 0)),
        out_shape=jax.ShapeDtypeStruct((2, 64, _HW), jnp.float32),
        scratch_shapes=[
            pltpu.VMEM((16, _HW), jnp.float32),
            pltpu.VMEM((16, _HW), jnp.float32),
            pltpu.VMEM((14, 16, _HW), jnp.bfloat16),
            pltpu.VMEM((14, 16, _HW), jnp.bfloat16),
            pltpu.VMEM((12, 32, _HW), jnp.bfloat16),
            pltpu.VMEM((12, 32, _HW), jnp.bfloat16),
            pltpu.VMEM((10, 16, _HW), jnp.bfloat16),
            pltpu.VMEM((10, 16, _HW), jnp.bfloat16),
            pltpu.VMEM((1152, _HW), jnp.bfloat16),
            pltpu.VMEM((1152, _HW), jnp.bfloat16),
        ],
        compiler_params=pltpu.CompilerParams(
            dimension_semantics=("arbitrary", "arbitrary")),
    )(*args)

    res = out.reshape(2, 64, 72, 72)[:, :, 4:68, 4:68]
    return res.reshape(2, 1, 64, 64, 64)
